# Initial kernel scaffold; baseline (speedup 1.0000x reference)
import jax
import jax.numpy as jnp
from jax.experimental import pallas as pl


def kernel(x, W_in, b_in, W_ih, b_ih, W_hh, b_hh, W_js, b_js, W_head, b_head):
    raise NotImplementedError("write your pallas kernel here")


# speedup vs baseline: 113.1260x; 113.1260x over previous
import jax
import jax.numpy as jnp
from jax.experimental import pallas as pl

_S = 8


def _fused_body(x0_ref, Win_ref, bin_ref, Wih_ref, bih_ref, bhh_ref,
                Whead_ref, bhead_ref, out_ref):
    x0 = x0_ref[...]
    inp = jax.lax.dot_general(
        x0, Win_ref[...], (((1,), (1,)), ((), ())),
        preferred_element_type=jnp.float32) + bin_ref[...]
    gi = jax.lax.dot_general(
        inp, Wih_ref[...], (((1,), (1,)), ((), ())),
        preferred_element_type=jnp.float32) + bih_ref[...]
    bhh = bhh_ref[...]
    r = jax.nn.sigmoid(gi[:, :_S] + bhh[:, :_S])
    z = jax.nn.sigmoid(gi[:, _S:2 * _S] + bhh[:, _S:2 * _S])
    n = jnp.tanh(gi[:, 2 * _S:] + r * bhh[:, 2 * _S:])
    upd = (1.0 - z) * n
    w8 = Whead_ref[...][:, :_S]
    out_ref[...] = jax.lax.dot_general(
        upd, w8, (((1,), (1,)), ((), ())),
        preferred_element_type=jnp.float32) + bhead_ref[...]


def kernel(x, W_in, b_in, W_ih, b_ih, W_hh, b_hh, W_js, b_js, W_head, b_head):
    Bq = x.shape[0]
    ncls = W_head.shape[0]
    x0 = x[:, 0, :]
    return pl.pallas_call(
        _fused_body,
        out_shape=jax.ShapeDtypeStruct((Bq, ncls), x.dtype),
    )(x0, W_in, b_in.reshape(1, -1), W_ih, b_ih.reshape(1, -1),
      b_hh.reshape(1, -1), W_head, b_head.reshape(1, -1))


# speedup vs baseline: 131.9161x; 1.1661x over previous
import jax
import jax.numpy as jnp
from jax.experimental import pallas as pl
from jax.experimental.pallas import tpu as pltpu

_S = 8


def _fused_body(x_hbm_ref, Win_ref, bin_ref, Wih_ref, bih_ref, bhh_ref,
                Whead_ref, bhead_ref, out_ref, x0_vmem, sem):
    cp = pltpu.make_async_copy(x_hbm_ref.at[:, 0, :], x0_vmem, sem)
    cp.start()
    cp.wait()
    x0 = x0_vmem[...]
    inp = jax.lax.dot_general(
        x0, Win_ref[...], (((1,), (1,)), ((), ())),
        preferred_element_type=jnp.float32) + bin_ref[...]
    gi = jax.lax.dot_general(
        inp, Wih_ref[...], (((1,), (1,)), ((), ())),
        preferred_element_type=jnp.float32) + bih_ref[...]
    bhh = bhh_ref[...]
    r = jax.nn.sigmoid(gi[:, :_S] + bhh[:, :_S])
    z = jax.nn.sigmoid(gi[:, _S:2 * _S] + bhh[:, _S:2 * _S])
    n = jnp.tanh(gi[:, 2 * _S:] + r * bhh[:, 2 * _S:])
    upd = (1.0 - z) * n
    w8 = Whead_ref[...][:, :_S]
    out_ref[...] = jax.lax.dot_general(
        upd, w8, (((1,), (1,)), ((), ())),
        preferred_element_type=jnp.float32) + bhead_ref[...]


def kernel(x, W_in, b_in, W_ih, b_ih, W_hh, b_hh, W_js, b_js, W_head, b_head):
    Bq, _, Dq = x.shape
    ncls = W_head.shape[0]
    return pl.pallas_call(
        _fused_body,
        out_shape=jax.ShapeDtypeStruct((Bq, ncls), x.dtype),
        in_specs=[
            pl.BlockSpec(memory_space=pltpu.MemorySpace.HBM),
            pl.BlockSpec(W_in.shape, lambda: (0, 0)),
            pl.BlockSpec((1, b_in.shape[0]), lambda: (0, 0)),
            pl.BlockSpec(W_ih.shape, lambda: (0, 0)),
            pl.BlockSpec((1, b_ih.shape[0]), lambda: (0, 0)),
            pl.BlockSpec((1, b_hh.shape[0]), lambda: (0, 0)),
            pl.BlockSpec(W_head.shape, lambda: (0, 0)),
            pl.BlockSpec((1, b_head.shape[0]), lambda: (0, 0)),
        ],
        out_specs=pl.BlockSpec((Bq, ncls), lambda: (0, 0)),
        scratch_shapes=[
            pltpu.VMEM((Bq, Dq), jnp.float32),
            pltpu.SemaphoreType.DMA,
        ],
    )(x, W_in, b_in.reshape(1, -1), W_ih, b_ih.reshape(1, -1),
      b_hh.reshape(1, -1), W_head, b_head.reshape(1, -1))
